# baseline (speedup 1.0000x reference)
import numpy as np
import jax
import jax.numpy as jnp
from jax import lax
from jax.experimental import pallas as pl
from jax.experimental.pallas import tpu as pltpu
from jax.experimental.pallas import tpu_sc as plsc

_DIM_VOCAB = 100000
_D = 1024
_H = _D // 2
_LEN_MAX = 2048
_BATCH = 8
_TOTAL = 8192

_NC, _NS, _L = 2, 16, 16
_NW = _NC * _NS
_TPW = _TOTAL // _NW
_CH = 8
_NCHUNK = _TPW // _CH
_NSLOT = 8
_LOOKAHEAD = 6
_NR = _NCHUNK // _NSLOT


def _pos_table():
    d = np.arange(_D)[np.newaxis, :]
    d = 1 / np.power(10000, 2 * (d // 2) / np.float32(_D))
    t = np.arange(_LEN_MAX)[:, np.newaxis] * d
    t = np.concatenate([np.sin(t[:, 0::2]), np.cos(t[:, 1::2])], axis=-1)
    return t.astype(np.float32)


_POS = _pos_table()


def _body(fv_hbm, rs_hbm, emb_hbm, pos_hbm, out_hbm,
          idx_v, rs_v, off_sp, g0, rotc, st, ebuf, *sems):
    gsems = list(sems[:_NSLOT])
    ssems = list(sems[_NSLOT:2 * _NSLOT])
    isem = sems[2 * _NSLOT]
    wid = lax.axis_index("s") * _NC + lax.axis_index("c")
    base = wid * _TPW

    pltpu.sync_copy(fv_hbm.at[pl.ds(base, _TPW)], idx_v)
    pltpu.sync_copy(rs_hbm, rs_v)
    pltpu.sync_copy(pos_hbm.at[pl.ds(1, 1)], rotc)

    @pl.loop(0, _TPW)
    def _(l):
        tok = jnp.full((_L,), base + l, jnp.int32)
        bvec = jnp.zeros((_L,), jnp.int32)
        for j in range(1, _BATCH + 1):
            rsj = rs_v[j]
            bvec = jnp.where(rsj <= tok, rsj, bvec)
        off_sp[l] = tok - bvec

    pltpu.async_copy(pos_hbm.at[off_sp.at[0].at[pl.ds(0, 1)]], g0, isem).wait()

    @pl.loop(0, _H // _L)
    def _(i):
        k = i * _L
        s = g0[0, pl.ds(k, _L)]
        c_ = g0[0, pl.ds(_H + k, _L)]
        sB = rotc[0, pl.ds(k, _L)]
        cB = rotc[0, pl.ds(_H + k, _L)]
        st[pl.ds(k, _L)] = s * cB - c_ * sB
        st[pl.ds(_H + k, _L)] = c_ * cB + s * sB

    def fire_gather(c, e):
        pltpu.async_copy(emb_hbm.at[idx_v.at[pl.ds(c * _CH, _CH)]],
                         ebuf.at[e], gsems[e])

    def drain_gather(e):
        pltpu.make_async_copy(emb_hbm.at[pl.ds(0, _CH)], ebuf.at[e],
                              gsems[e]).wait()

    def drain_store(e):
        pltpu.make_async_copy(ebuf.at[e], out_hbm.at[pl.ds(0, _CH)],
                              ssems[e]).wait()

    for c in range(_LOOKAHEAD):
        fire_gather(c, c)

    zero = jnp.zeros((_L,), jnp.float32)
    one = jnp.ones((_L,), jnp.float32)

    def chunk_add(e, c):
        for p in range(_H // (4 * _L)):
            kb = p * (4 * _L)
            sB = [rotc[0, pl.ds(kb + u * _L, _L)] for u in range(4)]
            cB = [rotc[0, pl.ds(_H + kb + u * _L, _L)] for u in range(4)]
            init = tuple(st[pl.ds(kb + u * _L, _L)] for u in range(4)) \
                + tuple(st[pl.ds(_H + kb + u * _L, _L)] for u in range(4))

            def tbody(t, carry, e=e, c=c, kb=kb, sB=sB, cB=cB):
                rmask = off_sp[c * _CH + t] == 0
                out = []
                for u in range(4):
                    s, c_ = carry[u], carry[4 + u]
                    ns = jnp.where(rmask, zero, s * cB[u] + c_ * sB[u])
                    nc = jnp.where(rmask, one, c_ * cB[u] - s * sB[u])
                    plsc.addupdate(ebuf.at[e, t, pl.ds(kb + u * _L, _L)], ns)
                    plsc.addupdate(
                        ebuf.at[e, t, pl.ds(_H + kb + u * _L, _L)], nc)
                    out.append((ns, nc))
                return tuple(v[0] for v in out) + tuple(v[1] for v in out)

            fin = pl.loop(0, _CH, init_carry=init)(tbody)
            for u in range(4):
                st[pl.ds(kb + u * _L, _L)] = fin[u]
                st[pl.ds(_H + kb + u * _L, _L)] = fin[4 + u]

    @pl.loop(0, _NR)
    def _(r):
        for e in range(_NSLOT):
            c = r * _NSLOT + e
            drain_gather(e)
            chunk_add(e, c)

            pltpu.async_copy(ebuf.at[e], out_hbm.at[pl.ds(base + c * _CH, _CH)],
                             ssems[e])

            cf = c + _LOOKAHEAD
            ef = (e + _LOOKAHEAD) % _NSLOT

            @pl.when(jnp.logical_and(cf >= _NSLOT, cf < _NCHUNK))
            def _():
                drain_store(ef)

            @pl.when(cf < _NCHUNK)
            def _():
                fire_gather(cf, ef)

    for e in range(_NSLOT):
        drain_store(e)


def kernel(fv, rs, emb):
    pos = jnp.asarray(_POS)
    rs16 = jnp.pad(rs, (0, _L - rs.shape[0]), mode="edge")
    rs_b = jnp.broadcast_to(rs16[:, None], (_L, _L))
    mesh = plsc.VectorSubcoreMesh(
        core_axis_name="c", subcore_axis_name="s",
        num_cores=_NC, num_subcores=_NS,
    )
    k = pl.kernel(
        _body,
        out_type=jax.ShapeDtypeStruct((_TOTAL, _D), jnp.float32),
        mesh=mesh,
        scratch_types=[
            pltpu.VMEM((_TPW,), jnp.int32),
            pltpu.VMEM((_L, _L), jnp.int32),
            pltpu.VMEM((_TPW, _L), jnp.int32),
            pltpu.VMEM((1, _D), jnp.float32),
            pltpu.VMEM((1, _D), jnp.float32),
            pltpu.VMEM((_D,), jnp.float32),
            pltpu.VMEM((_NSLOT, _CH, _D), jnp.float32),
        ] + [pltpu.SemaphoreType.DMA] * (2 * _NSLOT + 1),
    )
    return k(fv, rs_b, emb, pos)


# speedup vs baseline: 1.0726x; 1.0726x over previous
import numpy as np
import jax
import jax.numpy as jnp
from jax import lax
from jax.experimental import pallas as pl
from jax.experimental.pallas import tpu as pltpu
from jax.experimental.pallas import tpu_sc as plsc

_DIM_VOCAB = 100000
_D = 1024
_H = _D // 2
_LEN_MAX = 2048
_BATCH = 8
_TOTAL = 8192

_NC, _NS, _L = 2, 16, 16
_NW = _NC * _NS
_TPW = _TOTAL // _NW
_CH = 8
_NCHUNK = _TPW // _CH
_NSLOT = 4
_LOOKAHEAD = 3
_NR = _NCHUNK // _NSLOT


def _pos_table():
    d = np.arange(_D)[np.newaxis, :]
    d = 1 / np.power(10000, 2 * (d // 2) / np.float32(_D))
    t = np.arange(_LEN_MAX)[:, np.newaxis] * d
    t = np.concatenate([np.sin(t[:, 0::2]), np.cos(t[:, 1::2])], axis=-1)
    return t.astype(np.float32)


_POS = _pos_table()


def _body(fv_hbm, rs_hbm, emb_hbm, pos_hbm, out_hbm,
          idx_v, rs_v, off_sp, g0, rotc, st, ebuf, *sems):
    gsems = list(sems[:_NSLOT])
    ssems = list(sems[_NSLOT:2 * _NSLOT])
    isem = sems[2 * _NSLOT]
    wid = lax.axis_index("s") * _NC + lax.axis_index("c")
    base = wid * _TPW

    pltpu.sync_copy(fv_hbm.at[pl.ds(base, _TPW)], idx_v)

    for c0 in range(_LOOKAHEAD):
        pltpu.async_copy(emb_hbm.at[idx_v.at[pl.ds(c0 * _CH, _CH)]],
                         ebuf.at[c0], sems[c0])

    pltpu.sync_copy(rs_hbm, rs_v)
    pltpu.sync_copy(pos_hbm.at[pl.ds(1, 1)], rotc)

    @pl.loop(0, _TPW)
    def _(l):
        tok = jnp.full((_L,), base + l, jnp.int32)
        bvec = jnp.zeros((_L,), jnp.int32)
        for j in range(1, _BATCH + 1):
            rsj = rs_v[j]
            bvec = jnp.where(rsj <= tok, rsj, bvec)
        off_sp[l] = tok - bvec

    pltpu.async_copy(pos_hbm.at[off_sp.at[0].at[pl.ds(0, 1)]], g0, isem).wait()

    @pl.loop(0, _H // _L)
    def _(i):
        k = i * _L
        s = g0[0, pl.ds(k, _L)]
        c_ = g0[0, pl.ds(_H + k, _L)]
        sB = rotc[0, pl.ds(k, _L)]
        cB = rotc[0, pl.ds(_H + k, _L)]
        st[pl.ds(k, _L)] = s * cB - c_ * sB
        st[pl.ds(_H + k, _L)] = c_ * cB + s * sB

    def fire_gather(c, e):
        pltpu.async_copy(emb_hbm.at[idx_v.at[pl.ds(c * _CH, _CH)]],
                         ebuf.at[e], gsems[e])

    def drain_gather(e):
        pltpu.make_async_copy(emb_hbm.at[pl.ds(0, _CH)], ebuf.at[e],
                              gsems[e]).wait()

    def drain_store(e):
        pltpu.make_async_copy(ebuf.at[e], out_hbm.at[pl.ds(0, _CH)],
                              ssems[e]).wait()

    zero = jnp.zeros((_L,), jnp.float32)
    one = jnp.ones((_L,), jnp.float32)

    def chunk_add(e, c):
        for p in range(_H // (4 * _L)):
            kb = p * (4 * _L)
            sB = [rotc[0, pl.ds(kb + u * _L, _L)] for u in range(4)]
            cB = [rotc[0, pl.ds(_H + kb + u * _L, _L)] for u in range(4)]
            init = tuple(st[pl.ds(kb + u * _L, _L)] for u in range(4)) \
                + tuple(st[pl.ds(_H + kb + u * _L, _L)] for u in range(4))

            def tbody(t, carry, e=e, c=c, kb=kb, sB=sB, cB=cB):
                rmask = off_sp[c * _CH + t] == 0
                out = []
                for u in range(4):
                    s, c_ = carry[u], carry[4 + u]
                    ns = jnp.where(rmask, zero, s * cB[u] + c_ * sB[u])
                    nc = jnp.where(rmask, one, c_ * cB[u] - s * sB[u])
                    plsc.addupdate(ebuf.at[e, t, pl.ds(kb + u * _L, _L)], ns)
                    plsc.addupdate(
                        ebuf.at[e, t, pl.ds(_H + kb + u * _L, _L)], nc)
                    out.append((ns, nc))
                return tuple(v[0] for v in out) + tuple(v[1] for v in out)

            fin = pl.loop(0, _CH, init_carry=init)(tbody)
            for u in range(4):
                st[pl.ds(kb + u * _L, _L)] = fin[u]
                st[pl.ds(_H + kb + u * _L, _L)] = fin[4 + u]

    @pl.loop(0, _NR)
    def _(r):
        for e in range(_NSLOT):
            c = r * _NSLOT + e
            drain_gather(e)
            chunk_add(e, c)

            pltpu.async_copy(ebuf.at[e], out_hbm.at[pl.ds(base + c * _CH, _CH)],
                             ssems[e])

            cf = c + _LOOKAHEAD
            ef = (e + _LOOKAHEAD) % _NSLOT

            @pl.when(jnp.logical_and(cf >= _NSLOT, cf < _NCHUNK))
            def _():
                drain_store(ef)

            @pl.when(cf < _NCHUNK)
            def _():
                fire_gather(cf, ef)

    for e in range(_NSLOT):
        drain_store(e)


def kernel(fv, rs, emb):
    pos = jnp.asarray(_POS)
    rs16 = jnp.pad(rs, (0, _L - rs.shape[0]), mode="edge")
    rs_b = jnp.broadcast_to(rs16[:, None], (_L, _L))
    mesh = plsc.VectorSubcoreMesh(
        core_axis_name="c", subcore_axis_name="s",
        num_cores=_NC, num_subcores=_NS,
    )
    k = pl.kernel(
        _body,
        out_type=jax.ShapeDtypeStruct((_TOTAL, _D), jnp.float32),
        mesh=mesh,
        scratch_types=[
            pltpu.VMEM((_TPW,), jnp.int32),
            pltpu.VMEM((_L, _L), jnp.int32),
            pltpu.VMEM((_TPW, _L), jnp.int32),
            pltpu.VMEM((1, _D), jnp.float32),
            pltpu.VMEM((1, _D), jnp.float32),
            pltpu.VMEM((_D,), jnp.float32),
            pltpu.VMEM((_NSLOT, _CH, _D), jnp.float32),
        ] + [pltpu.SemaphoreType.DMA] * (2 * _NSLOT + 1),
    )
    return k(fv, rs_b, emb, pos)


# speedup vs baseline: 1.1743x; 1.0948x over previous
import numpy as np
import jax
import jax.numpy as jnp
from jax import lax
from jax.experimental import pallas as pl
from jax.experimental.pallas import tpu as pltpu
from jax.experimental.pallas import tpu_sc as plsc

_DIM_VOCAB = 100000
_D = 1024
_H = _D // 2
_LEN_MAX = 2048
_BATCH = 8
_TOTAL = 8192

_NC, _NS, _L = 2, 16, 16
_NW = _NC * _NS
_TPW = _TOTAL // _NW
_CH = 8
_NCHUNK = _TPW // _CH
_NSLOT = 4
_LOOKAHEAD = 3
_NBIT = 11
_NR = _NCHUNK // _NSLOT


def _pos_table():
    d = np.arange(_D)[np.newaxis, :]
    d = 1 / np.power(10000, 2 * (d // 2) / np.float32(_D))
    t = np.arange(_LEN_MAX)[:, np.newaxis] * d
    t = np.concatenate([np.sin(t[:, 0::2]), np.cos(t[:, 1::2])], axis=-1)
    return t.astype(np.float32)


_POS = _pos_table()


def _body(fv_hbm, rs_hbm, emb_hbm, ptab_hbm, out_hbm,
          idx_v, rs_v, off_sp, ptab, st, ebuf, *sems):
    gsems = list(sems[:_NSLOT])
    ssems = list(sems[_NSLOT:2 * _NSLOT])
    wid = lax.axis_index("s") * _NC + lax.axis_index("c")
    base = wid * _TPW

    pltpu.sync_copy(fv_hbm.at[pl.ds(base, _TPW)], idx_v)

    for c0 in range(_LOOKAHEAD):
        pltpu.async_copy(emb_hbm.at[idx_v.at[pl.ds(c0 * _CH, _CH)]],
                         ebuf.at[c0], sems[c0])

    pltpu.sync_copy(rs_hbm, rs_v)
    pltpu.sync_copy(ptab_hbm, ptab)

    @pl.loop(0, _TPW)
    def _(l):
        tok = jnp.full((_L,), base + l, jnp.int32)
        bvec = jnp.zeros((_L,), jnp.int32)
        for j in range(1, _BATCH + 1):
            rsj = rs_v[j]
            bvec = jnp.where(rsj <= tok, rsj, bvec)
        off_sp[l] = tok - bvec

    off0 = off_sp[0]
    bitm = [(jnp.right_shift(off0, k) & 1) == 1 for k in range(_NBIT)]

    @pl.loop(0, _H // _L)
    def _(i):
        k = i * _L
        s = jnp.zeros((_L,), jnp.float32)
        c_ = jnp.ones((_L,), jnp.float32)
        for kk in range(_NBIT):
            sB = ptab[1 + kk, pl.ds(k, _L)]
            cB = ptab[1 + kk, pl.ds(_H + k, _L)]
            s, c_ = (jnp.where(bitm[kk], s * cB + c_ * sB, s),
                     jnp.where(bitm[kk], c_ * cB - s * sB, c_))
        sB = ptab[0, pl.ds(k, _L)]
        cB = ptab[0, pl.ds(_H + k, _L)]
        st[pl.ds(k, _L)] = s * cB - c_ * sB
        st[pl.ds(_H + k, _L)] = c_ * cB + s * sB

    def fire_gather(c, e):
        pltpu.async_copy(emb_hbm.at[idx_v.at[pl.ds(c * _CH, _CH)]],
                         ebuf.at[e], gsems[e])

    def drain_gather(e):
        pltpu.make_async_copy(emb_hbm.at[pl.ds(0, _CH)], ebuf.at[e],
                              gsems[e]).wait()

    def drain_store(e):
        pltpu.make_async_copy(ebuf.at[e], out_hbm.at[pl.ds(0, _CH)],
                              ssems[e]).wait()

    zero = jnp.zeros((_L,), jnp.float32)
    one = jnp.ones((_L,), jnp.float32)

    def chunk_add(e, c):
        for p in range(_H // (4 * _L)):
            kb = p * (4 * _L)
            sB = [ptab[0, pl.ds(kb + u * _L, _L)] for u in range(4)]
            cB = [ptab[0, pl.ds(_H + kb + u * _L, _L)] for u in range(4)]
            init = tuple(st[pl.ds(kb + u * _L, _L)] for u in range(4)) \
                + tuple(st[pl.ds(_H + kb + u * _L, _L)] for u in range(4))

            def tbody(t, carry, e=e, c=c, kb=kb, sB=sB, cB=cB):
                rmask = off_sp[c * _CH + t] == 0
                out = []
                for u in range(4):
                    s, c_ = carry[u], carry[4 + u]
                    ns = jnp.where(rmask, zero, s * cB[u] + c_ * sB[u])
                    nc = jnp.where(rmask, one, c_ * cB[u] - s * sB[u])
                    plsc.addupdate(ebuf.at[e, t, pl.ds(kb + u * _L, _L)], ns)
                    plsc.addupdate(
                        ebuf.at[e, t, pl.ds(_H + kb + u * _L, _L)], nc)
                    out.append((ns, nc))
                return tuple(v[0] for v in out) + tuple(v[1] for v in out)

            fin = pl.loop(0, _CH, init_carry=init)(tbody)
            for u in range(4):
                st[pl.ds(kb + u * _L, _L)] = fin[u]
                st[pl.ds(_H + kb + u * _L, _L)] = fin[4 + u]

    @pl.loop(0, _NR)
    def _(r):
        for e in range(_NSLOT):
            c = r * _NSLOT + e
            drain_gather(e)
            chunk_add(e, c)

            pltpu.async_copy(ebuf.at[e], out_hbm.at[pl.ds(base + c * _CH, _CH)],
                             ssems[e])

            cf = c + _LOOKAHEAD
            ef = (e + _LOOKAHEAD) % _NSLOT

            @pl.when(jnp.logical_and(cf >= _NSLOT, cf < _NCHUNK))
            def _():
                drain_store(ef)

            @pl.when(cf < _NCHUNK)
            def _():
                fire_gather(cf, ef)

    for e in range(_NSLOT):
        drain_store(e)


def kernel(fv, rs, emb):
    ptab_np = np.concatenate(
        [_POS[1:2]] + [_POS[2 ** k:2 ** k + 1] for k in range(_NBIT)])
    ptab = jnp.asarray(ptab_np)
    rs16 = jnp.pad(rs, (0, _L - rs.shape[0]), mode="edge")
    rs_b = jnp.broadcast_to(rs16[:, None], (_L, _L))
    mesh = plsc.VectorSubcoreMesh(
        core_axis_name="c", subcore_axis_name="s",
        num_cores=_NC, num_subcores=_NS,
    )
    k = pl.kernel(
        _body,
        out_type=jax.ShapeDtypeStruct((_TOTAL, _D), jnp.float32),
        mesh=mesh,
        scratch_types=[
            pltpu.VMEM((_TPW,), jnp.int32),
            pltpu.VMEM((_L, _L), jnp.int32),
            pltpu.VMEM((_TPW, _L), jnp.int32),
            pltpu.VMEM((1 + _NBIT, _D), jnp.float32),
            pltpu.VMEM((_D,), jnp.float32),
            pltpu.VMEM((_NSLOT, _CH, _D), jnp.float32),
        ] + [pltpu.SemaphoreType.DMA] * (2 * _NSLOT),
    )
    return k(fv, rs_b, emb, ptab)


# speedup vs baseline: 1.2687x; 1.0804x over previous
import numpy as np
import jax
import jax.numpy as jnp
from jax import lax
from jax.experimental import pallas as pl
from jax.experimental.pallas import tpu as pltpu
from jax.experimental.pallas import tpu_sc as plsc

_DIM_VOCAB = 100000
_D = 1024
_H = _D // 2
_LEN_MAX = 2048
_BATCH = 8
_TOTAL = 8192

_NC, _NS, _L = 2, 16, 16
_NW = _NC * _NS
_TPW = _TOTAL // _NW
_CH = 16
_NCHUNK = _TPW // _CH
_NSLOT = 4
_LOOKAHEAD = 3
_NBIT = 11
_NR = _NCHUNK // _NSLOT


def _pos_table():
    d = np.arange(_D)[np.newaxis, :]
    d = 1 / np.power(10000, 2 * (d // 2) / np.float32(_D))
    t = np.arange(_LEN_MAX)[:, np.newaxis] * d
    t = np.concatenate([np.sin(t[:, 0::2]), np.cos(t[:, 1::2])], axis=-1)
    return t.astype(np.float32)


_POS = _pos_table()


def _body(fv_hbm, rs_hbm, emb_hbm, ptab_hbm, out_hbm,
          idx_v, rs_v, off_sp, ptab, st, ebuf, *sems):
    gsems = list(sems[:_NSLOT])
    ssems = list(sems[_NSLOT:2 * _NSLOT])
    wid = lax.axis_index("s") * _NC + lax.axis_index("c")
    base = wid * _TPW

    pltpu.sync_copy(fv_hbm.at[pl.ds(base, _TPW)], idx_v)

    for c0 in range(_LOOKAHEAD):
        pltpu.async_copy(emb_hbm.at[idx_v.at[pl.ds(c0 * _CH, _CH)]],
                         ebuf.at[c0], sems[c0])

    pltpu.sync_copy(rs_hbm, rs_v)
    pltpu.sync_copy(ptab_hbm, ptab)

    @pl.loop(0, _TPW)
    def _(l):
        tok = jnp.full((_L,), base + l, jnp.int32)
        bvec = jnp.zeros((_L,), jnp.int32)
        for j in range(1, _BATCH + 1):
            rsj = rs_v[j]
            bvec = jnp.where(rsj <= tok, rsj, bvec)
        off_sp[l] = tok - bvec

    off0 = off_sp[0]
    bitm = [(jnp.right_shift(off0, k) & 1) == 1 for k in range(_NBIT)]

    @pl.loop(0, _H // _L)
    def _(i):
        k = i * _L
        s = jnp.zeros((_L,), jnp.float32)
        c_ = jnp.ones((_L,), jnp.float32)
        for kk in range(_NBIT):
            sB = ptab[1 + kk, pl.ds(k, _L)]
            cB = ptab[1 + kk, pl.ds(_H + k, _L)]
            s, c_ = (jnp.where(bitm[kk], s * cB + c_ * sB, s),
                     jnp.where(bitm[kk], c_ * cB - s * sB, c_))
        sB = ptab[0, pl.ds(k, _L)]
        cB = ptab[0, pl.ds(_H + k, _L)]
        st[pl.ds(k, _L)] = s * cB - c_ * sB
        st[pl.ds(_H + k, _L)] = c_ * cB + s * sB

    def fire_gather(c, e):
        pltpu.async_copy(emb_hbm.at[idx_v.at[pl.ds(c * _CH, _CH)]],
                         ebuf.at[e], gsems[e])

    def drain_gather(e):
        pltpu.make_async_copy(emb_hbm.at[pl.ds(0, _CH)], ebuf.at[e],
                              gsems[e]).wait()

    def drain_store(e):
        pltpu.make_async_copy(ebuf.at[e], out_hbm.at[pl.ds(0, _CH)],
                              ssems[e]).wait()

    zero = jnp.zeros((_L,), jnp.float32)
    one = jnp.ones((_L,), jnp.float32)

    def chunk_add(e, c):
        for p in range(_H // (4 * _L)):
            kb = p * (4 * _L)
            sB = [ptab[0, pl.ds(kb + u * _L, _L)] for u in range(4)]
            cB = [ptab[0, pl.ds(_H + kb + u * _L, _L)] for u in range(4)]
            init = tuple(st[pl.ds(kb + u * _L, _L)] for u in range(4)) \
                + tuple(st[pl.ds(_H + kb + u * _L, _L)] for u in range(4))

            def tbody(t, carry, e=e, c=c, kb=kb, sB=sB, cB=cB):
                rmask = off_sp[c * _CH + t] == 0
                out = []
                for u in range(4):
                    s, c_ = carry[u], carry[4 + u]
                    ns = jnp.where(rmask, zero, s * cB[u] + c_ * sB[u])
                    nc = jnp.where(rmask, one, c_ * cB[u] - s * sB[u])
                    plsc.addupdate(ebuf.at[e, t, pl.ds(kb + u * _L, _L)], ns)
                    plsc.addupdate(
                        ebuf.at[e, t, pl.ds(_H + kb + u * _L, _L)], nc)
                    out.append((ns, nc))
                return tuple(v[0] for v in out) + tuple(v[1] for v in out)

            fin = pl.loop(0, _CH, init_carry=init)(tbody)
            for u in range(4):
                st[pl.ds(kb + u * _L, _L)] = fin[u]
                st[pl.ds(_H + kb + u * _L, _L)] = fin[4 + u]

    @pl.loop(0, _NR)
    def _(r):
        for e in range(_NSLOT):
            c = r * _NSLOT + e
            drain_gather(e)
            chunk_add(e, c)

            pltpu.async_copy(ebuf.at[e], out_hbm.at[pl.ds(base + c * _CH, _CH)],
                             ssems[e])

            cf = c + _LOOKAHEAD
            ef = (e + _LOOKAHEAD) % _NSLOT

            @pl.when(jnp.logical_and(cf >= _NSLOT, cf < _NCHUNK))
            def _():
                drain_store(ef)

            @pl.when(cf < _NCHUNK)
            def _():
                fire_gather(cf, ef)

    for e in range(_NSLOT):
        drain_store(e)


def kernel(fv, rs, emb):
    ptab_np = np.concatenate(
        [_POS[1:2]] + [_POS[2 ** k:2 ** k + 1] for k in range(_NBIT)])
    ptab = jnp.asarray(ptab_np)
    rs16 = jnp.pad(rs, (0, _L - rs.shape[0]), mode="edge")
    rs_b = jnp.broadcast_to(rs16[:, None], (_L, _L))
    mesh = plsc.VectorSubcoreMesh(
        core_axis_name="c", subcore_axis_name="s",
        num_cores=_NC, num_subcores=_NS,
    )
    k = pl.kernel(
        _body,
        out_type=jax.ShapeDtypeStruct((_TOTAL, _D), jnp.float32),
        mesh=mesh,
        scratch_types=[
            pltpu.VMEM((_TPW,), jnp.int32),
            pltpu.VMEM((_L, _L), jnp.int32),
            pltpu.VMEM((_TPW, _L), jnp.int32),
            pltpu.VMEM((1 + _NBIT, _D), jnp.float32),
            pltpu.VMEM((_D,), jnp.float32),
            pltpu.VMEM((_NSLOT, _CH, _D), jnp.float32),
        ] + [pltpu.SemaphoreType.DMA] * (2 * _NSLOT),
    )
    return k(fv, rs_b, emb, ptab)


# speedup vs baseline: 1.2692x; 1.0004x over previous
import numpy as np
import jax
import jax.numpy as jnp
from jax import lax
from jax.experimental import pallas as pl
from jax.experimental.pallas import tpu as pltpu
from jax.experimental.pallas import tpu_sc as plsc

_DIM_VOCAB = 100000
_D = 1024
_H = _D // 2
_LEN_MAX = 2048
_BATCH = 8
_TOTAL = 8192

_NC, _NS, _L = 2, 16, 16
_NW = _NC * _NS
_TPW = _TOTAL // _NW
_CH = 16
_NCHUNK = _TPW // _CH
_NSLOT = 4
_LOOKAHEAD = 3
_NBIT = 11
_NR = _NCHUNK // _NSLOT


def _pos_table():
    d = np.arange(_D)[np.newaxis, :]
    d = 1 / np.power(10000, 2 * (d // 2) / np.float32(_D))
    t = np.arange(_LEN_MAX)[:, np.newaxis] * d
    t = np.concatenate([np.sin(t[:, 0::2]), np.cos(t[:, 1::2])], axis=-1)
    return t.astype(np.float32)


_POS = _pos_table()


def _body(fv_hbm, rs_hbm, emb_hbm, ptab_hbm, out_hbm,
          idx_v, rs_v, off_sp, ptab, st, ebuf, *sems):
    gsems = list(sems[:_NSLOT])
    ssems = list(sems[_NSLOT:2 * _NSLOT])
    wid = lax.axis_index("s") * _NC + lax.axis_index("c")
    base = wid * _TPW

    pltpu.sync_copy(fv_hbm.at[pl.ds(base, _TPW)], idx_v)

    for c0 in range(_LOOKAHEAD):
        pltpu.async_copy(emb_hbm.at[idx_v.at[pl.ds(c0 * _CH, _CH)]],
                         ebuf.at[c0], sems[c0])

    pltpu.sync_copy(rs_hbm, rs_v)
    pltpu.sync_copy(ptab_hbm, ptab)

    @pl.loop(0, _TPW)
    def _(l):
        tok = jnp.full((_L,), base + l, jnp.int32)
        bvec = jnp.zeros((_L,), jnp.int32)
        for j in range(1, _BATCH + 1):
            rsj = rs_v[j]
            bvec = jnp.where(rsj <= tok, rsj, bvec)
        off_sp[l] = tok - bvec

    off0 = off_sp[0]
    bitm = [(jnp.right_shift(off0, k) & 1) == 1 for k in range(_NBIT)]

    @pl.loop(0, _H // _L)
    def _(i):
        k = i * _L
        s = jnp.zeros((_L,), jnp.float32)
        c_ = jnp.ones((_L,), jnp.float32)
        for kk in range(_NBIT):
            sB = ptab[1 + kk, pl.ds(k, _L)]
            cB = ptab[1 + kk, pl.ds(_H + k, _L)]
            s, c_ = (jnp.where(bitm[kk], s * cB + c_ * sB, s),
                     jnp.where(bitm[kk], c_ * cB - s * sB, c_))
        sB = ptab[0, pl.ds(k, _L)]
        cB = ptab[0, pl.ds(_H + k, _L)]
        st[pl.ds(k, _L)] = s * cB - c_ * sB
        st[pl.ds(_H + k, _L)] = c_ * cB + s * sB

    def fire_gather(c, e):
        pltpu.async_copy(emb_hbm.at[idx_v.at[pl.ds(c * _CH, _CH)]],
                         ebuf.at[e], gsems[e])

    def drain_gather(e):
        pltpu.make_async_copy(emb_hbm.at[pl.ds(0, _CH)], ebuf.at[e],
                              gsems[e]).wait()

    def drain_store(e):
        pltpu.make_async_copy(ebuf.at[e], out_hbm.at[pl.ds(0, _CH)],
                              ssems[e]).wait()

    zero = jnp.zeros((_L,), jnp.float32)
    one = jnp.ones((_L,), jnp.float32)

    def chunk_add(e, c):
        for p in range(_H // (4 * _L)):
            kb = p * (4 * _L)
            sB = [ptab[0, pl.ds(kb + u * _L, _L)] for u in range(4)]
            cB = [ptab[0, pl.ds(_H + kb + u * _L, _L)] for u in range(4)]
            init = tuple(st[pl.ds(kb + u * _L, _L)] for u in range(4)) \
                + tuple(st[pl.ds(_H + kb + u * _L, _L)] for u in range(4))

            def tbody(t, carry, e=e, c=c, kb=kb, sB=sB, cB=cB):
                rmask = off_sp[c * _CH + t] == 0
                out = []
                for u in range(4):
                    s, c_ = carry[u], carry[4 + u]
                    ns = jnp.where(rmask, zero, s * cB[u] + c_ * sB[u])
                    nc = jnp.where(rmask, one, c_ * cB[u] - s * sB[u])
                    plsc.addupdate(ebuf.at[e, t, pl.ds(kb + u * _L, _L)], ns)
                    plsc.addupdate(
                        ebuf.at[e, t, pl.ds(_H + kb + u * _L, _L)], nc)
                    out.append((ns, nc))
                return tuple(v[0] for v in out) + tuple(v[1] for v in out)

            fin = pl.loop(0, _CH, init_carry=init)(tbody)
            for u in range(4):
                st[pl.ds(kb + u * _L, _L)] = fin[u]
                st[pl.ds(_H + kb + u * _L, _L)] = fin[4 + u]

    @pl.loop(0, _NR)
    def _(r):
        for e in range(_NSLOT):
            c = r * _NSLOT + e
            drain_gather(e)
            chunk_add(e, c)

            pltpu.async_copy(ebuf.at[e], out_hbm.at[pl.ds(base + c * _CH, _CH)],
                             ssems[e])

            cf = c + _LOOKAHEAD
            ef = (e + _LOOKAHEAD) % _NSLOT

            @pl.when(jnp.logical_and(cf >= _NSLOT, cf < _NCHUNK))
            def _():
                drain_store(ef)

            @pl.when(cf < _NCHUNK)
            def _():
                fire_gather(cf, ef)

    for e in range(_NSLOT):
        drain_store(e)


def kernel(fv, rs, emb):
    ptab_np = np.concatenate(
        [_POS[1:2]] + [_POS[2 ** k:2 ** k + 1] for k in range(_NBIT)])
    ptab = jnp.asarray(ptab_np)
    rs_b = jnp.broadcast_to(rs[:, None], (_BATCH + 1, _L))
    mesh = plsc.VectorSubcoreMesh(
        core_axis_name="c", subcore_axis_name="s",
        num_cores=_NC, num_subcores=_NS,
    )
    k = pl.kernel(
        _body,
        out_type=jax.ShapeDtypeStruct((_TOTAL, _D), jnp.float32),
        mesh=mesh,
        scratch_types=[
            pltpu.VMEM((_TPW,), jnp.int32),
            pltpu.VMEM((_BATCH + 1, _L), jnp.int32),
            pltpu.VMEM((_TPW, _L), jnp.int32),
            pltpu.VMEM((1 + _NBIT, _D), jnp.float32),
            pltpu.VMEM((_D,), jnp.float32),
            pltpu.VMEM((_NSLOT, _CH, _D), jnp.float32),
        ] + [pltpu.SemaphoreType.DMA] * (2 * _NSLOT),
    )
    return k(fv, rs_b, emb, ptab)


# speedup vs baseline: 1.2698x; 1.0004x over previous
import numpy as np
import jax
import jax.numpy as jnp
from jax import lax
from jax.experimental import pallas as pl
from jax.experimental.pallas import tpu as pltpu
from jax.experimental.pallas import tpu_sc as plsc

_DIM_VOCAB = 100000
_D = 1024
_H = _D // 2
_LEN_MAX = 2048
_BATCH = 8
_TOTAL = 8192

_NC, _NS, _L = 2, 16, 16
_NW = _NC * _NS
_TPW = _TOTAL // _NW
_CH = 16
_NCHUNK = _TPW // _CH
_NSLOT = 4
_LOOKAHEAD = 3
_NBIT = 11
_NR = _NCHUNK // _NSLOT


def _pos_table():
    d = np.arange(_D)[np.newaxis, :]
    d = 1 / np.power(10000, 2 * (d // 2) / np.float32(_D))
    t = np.arange(_LEN_MAX)[:, np.newaxis] * d
    t = np.concatenate([np.sin(t[:, 0::2]), np.cos(t[:, 1::2])], axis=-1)
    return t.astype(np.float32)


_POS = _pos_table()


def _body(fv_hbm, rs_hbm, emb_hbm, ptab_hbm, out_hbm,
          idx_v, rs_v, off_sp, ptab, st, ebuf, *sems):
    gsems = list(sems[:_NSLOT])
    ssems = list(sems[_NSLOT:2 * _NSLOT])
    wid = lax.axis_index("s") * _NC + lax.axis_index("c")
    base = wid * _TPW

    pltpu.sync_copy(fv_hbm.at[pl.ds(base, _TPW)], idx_v)

    for c0 in range(_LOOKAHEAD):
        pltpu.async_copy(emb_hbm.at[idx_v.at[pl.ds(c0 * _CH, _CH)]],
                         ebuf.at[c0], sems[c0])

    pltpu.sync_copy(rs_hbm, rs_v)
    pltpu.sync_copy(ptab_hbm, ptab)

    @pl.loop(0, _TPW)
    def _(l):
        tok = jnp.full((_L,), base + l, jnp.int32)
        bvec = jnp.zeros((_L,), jnp.int32)
        for j in range(1, _BATCH + 1):
            rsj = rs_v[j]
            bvec = jnp.where(rsj <= tok, rsj, bvec)
        off_sp[l] = tok - bvec

    off0 = off_sp[0]
    bitm = [(jnp.right_shift(off0, k) & 1) == 1 for k in range(_NBIT)]

    @pl.loop(0, _H // _L)
    def _(i):
        k = i * _L
        s = jnp.zeros((_L,), jnp.float32)
        c_ = jnp.ones((_L,), jnp.float32)
        for kk in range(_NBIT):
            sB = ptab[1 + kk, pl.ds(k, _L)]
            cB = ptab[1 + kk, pl.ds(_H + k, _L)]
            s, c_ = (jnp.where(bitm[kk], s * cB + c_ * sB, s),
                     jnp.where(bitm[kk], c_ * cB - s * sB, c_))
        sB = ptab[0, pl.ds(k, _L)]
        cB = ptab[0, pl.ds(_H + k, _L)]
        st[pl.ds(k, _L)] = s * cB - c_ * sB
        st[pl.ds(_H + k, _L)] = c_ * cB + s * sB

    _HC = _CH // 2

    def fire_gather(c, e):
        pltpu.async_copy(emb_hbm.at[idx_v.at[pl.ds(c * _CH, _HC)]],
                         ebuf.at[e, pl.ds(0, _HC)], gsems[e])
        pltpu.async_copy(emb_hbm.at[idx_v.at[pl.ds(c * _CH + _HC, _HC)]],
                         ebuf.at[e, pl.ds(_HC, _HC)], gsems[e])

    def drain_gather(e):
        pltpu.make_async_copy(emb_hbm.at[pl.ds(0, _HC)],
                              ebuf.at[e, pl.ds(0, _HC)], gsems[e]).wait()
        pltpu.make_async_copy(emb_hbm.at[pl.ds(0, _HC)],
                              ebuf.at[e, pl.ds(_HC, _HC)], gsems[e]).wait()

    def drain_store(e):
        pltpu.make_async_copy(ebuf.at[e], out_hbm.at[pl.ds(0, _CH)],
                              ssems[e]).wait()

    zero = jnp.zeros((_L,), jnp.float32)
    one = jnp.ones((_L,), jnp.float32)

    def chunk_add(e, c):
        for p in range(_H // (4 * _L)):
            kb = p * (4 * _L)
            sB = [ptab[0, pl.ds(kb + u * _L, _L)] for u in range(4)]
            cB = [ptab[0, pl.ds(_H + kb + u * _L, _L)] for u in range(4)]
            init = tuple(st[pl.ds(kb + u * _L, _L)] for u in range(4)) \
                + tuple(st[pl.ds(_H + kb + u * _L, _L)] for u in range(4))

            def tbody(t, carry, e=e, c=c, kb=kb, sB=sB, cB=cB):
                rmask = off_sp[c * _CH + t] == 0
                out = []
                for u in range(4):
                    s, c_ = carry[u], carry[4 + u]
                    ns = jnp.where(rmask, zero, s * cB[u] + c_ * sB[u])
                    nc = jnp.where(rmask, one, c_ * cB[u] - s * sB[u])
                    plsc.addupdate(ebuf.at[e, t, pl.ds(kb + u * _L, _L)], ns)
                    plsc.addupdate(
                        ebuf.at[e, t, pl.ds(_H + kb + u * _L, _L)], nc)
                    out.append((ns, nc))
                return tuple(v[0] for v in out) + tuple(v[1] for v in out)

            fin = pl.loop(0, _CH, init_carry=init)(tbody)
            for u in range(4):
                st[pl.ds(kb + u * _L, _L)] = fin[u]
                st[pl.ds(_H + kb + u * _L, _L)] = fin[4 + u]

    @pl.loop(0, _NR)
    def _(r):
        for e in range(_NSLOT):
            c = r * _NSLOT + e
            drain_gather(e)
            chunk_add(e, c)

            pltpu.async_copy(ebuf.at[e], out_hbm.at[pl.ds(base + c * _CH, _CH)],
                             ssems[e])

            cf = c + _LOOKAHEAD
            ef = (e + _LOOKAHEAD) % _NSLOT

            @pl.when(jnp.logical_and(cf >= _NSLOT, cf < _NCHUNK))
            def _():
                drain_store(ef)

            @pl.when(cf < _NCHUNK)
            def _():
                fire_gather(cf, ef)

    for e in range(_NSLOT):
        drain_store(e)


def kernel(fv, rs, emb):
    ptab_np = np.concatenate(
        [_POS[1:2]] + [_POS[2 ** k:2 ** k + 1] for k in range(_NBIT)])
    ptab = jnp.asarray(ptab_np)
    rs_b = jnp.broadcast_to(rs[:, None], (_BATCH + 1, _L))
    mesh = plsc.VectorSubcoreMesh(
        core_axis_name="c", subcore_axis_name="s",
        num_cores=_NC, num_subcores=_NS,
    )
    k = pl.kernel(
        _body,
        out_type=jax.ShapeDtypeStruct((_TOTAL, _D), jnp.float32),
        mesh=mesh,
        scratch_types=[
            pltpu.VMEM((_TPW,), jnp.int32),
            pltpu.VMEM((_BATCH + 1, _L), jnp.int32),
            pltpu.VMEM((_TPW, _L), jnp.int32),
            pltpu.VMEM((1 + _NBIT, _D), jnp.float32),
            pltpu.VMEM((_D,), jnp.float32),
            pltpu.VMEM((_NSLOT, _CH, _D), jnp.float32),
        ] + [pltpu.SemaphoreType.DMA] * (2 * _NSLOT),
    )
    return k(fv, rs_b, emb, ptab)


# speedup vs baseline: 1.2721x; 1.0018x over previous
import numpy as np
import jax
import jax.numpy as jnp
from jax import lax
from jax.experimental import pallas as pl
from jax.experimental.pallas import tpu as pltpu
from jax.experimental.pallas import tpu_sc as plsc

_DIM_VOCAB = 100000
_D = 1024
_H = _D // 2
_LEN_MAX = 2048
_BATCH = 8
_TOTAL = 8192

_NC, _NS, _L = 2, 16, 16
_NW = _NC * _NS
_TPW = _TOTAL // _NW
_CH = 16
_NCHUNK = _TPW // _CH
_NSLOT = 4
_LOOKAHEAD = 3
_NBIT = 11
_NR = _NCHUNK // _NSLOT


def _pos_table():
    d = np.arange(_D)[np.newaxis, :]
    d = 1 / np.power(10000, 2 * (d // 2) / np.float32(_D))
    t = np.arange(_LEN_MAX)[:, np.newaxis] * d
    t = np.concatenate([np.sin(t[:, 0::2]), np.cos(t[:, 1::2])], axis=-1)
    return t.astype(np.float32)


_POS = _pos_table()


def _body(fv_hbm, rs_hbm, emb_hbm, ptab_hbm, out_hbm,
          idx_v, rs_v, off_sp, ptab, st, ebuf, *sems):
    gsems = list(sems[:_NSLOT])
    ssems = list(sems[_NSLOT:2 * _NSLOT])
    wid = lax.axis_index("s") * _NC + lax.axis_index("c")
    base = wid * _TPW

    pltpu.sync_copy(fv_hbm.at[pl.ds(base, _TPW)], idx_v)

    for c0 in range(_LOOKAHEAD):
        pltpu.async_copy(emb_hbm.at[idx_v.at[pl.ds(c0 * _CH, _CH)]],
                         ebuf.at[c0], sems[c0])

    pltpu.sync_copy(rs_hbm, rs_v)
    pltpu.sync_copy(ptab_hbm, ptab)

    @pl.loop(0, _TPW)
    def _(l):
        tok = jnp.full((_L,), base + l, jnp.int32)
        bvec = jnp.zeros((_L,), jnp.int32)
        for j in range(1, _BATCH + 1):
            rsj = rs_v[j]
            bvec = jnp.where(rsj <= tok, rsj, bvec)
        off_sp[l] = tok - bvec

    off0 = off_sp[0]
    bitm = [(jnp.right_shift(off0, k) & 1) == 1 for k in range(_NBIT)]

    @pl.loop(0, _H // _L)
    def _(i):
        k = i * _L
        s = jnp.zeros((_L,), jnp.float32)
        c_ = jnp.ones((_L,), jnp.float32)
        for kk in range(_NBIT):
            sB = ptab[1 + kk, pl.ds(k, _L)]
            cB = ptab[1 + kk, pl.ds(_H + k, _L)]
            s, c_ = (jnp.where(bitm[kk], s * cB + c_ * sB, s),
                     jnp.where(bitm[kk], c_ * cB - s * sB, c_))
        sB = ptab[0, pl.ds(k, _L)]
        cB = ptab[0, pl.ds(_H + k, _L)]
        st[pl.ds(k, _L)] = s * cB - c_ * sB
        st[pl.ds(_H + k, _L)] = c_ * cB + s * sB

    def fire_gather(c, e):
        pltpu.async_copy(emb_hbm.at[idx_v.at[pl.ds(c * _CH, _CH)]],
                         ebuf.at[e], gsems[e])

    def drain_gather(e):
        pltpu.make_async_copy(emb_hbm.at[pl.ds(0, _CH)], ebuf.at[e],
                              gsems[e]).wait()

    def drain_store(e):
        pltpu.make_async_copy(ebuf.at[e], out_hbm.at[pl.ds(0, _CH)],
                              ssems[e]).wait()

    zero = jnp.zeros((_L,), jnp.float32)
    one = jnp.ones((_L,), jnp.float32)

    def chunk_add(e, c):
        for p in range(_H // (4 * _L)):
            kb = p * (4 * _L)
            sB = [ptab[0, pl.ds(kb + u * _L, _L)] for u in range(4)]
            cB = [ptab[0, pl.ds(_H + kb + u * _L, _L)] for u in range(4)]
            init = tuple(st[pl.ds(kb + u * _L, _L)] for u in range(4)) \
                + tuple(st[pl.ds(_H + kb + u * _L, _L)] for u in range(4))

            def tbody(t, carry, e=e, c=c, kb=kb, sB=sB, cB=cB):
                rmask = off_sp[c * _CH + t] == 0
                out = []
                for u in range(4):
                    s, c_ = carry[u], carry[4 + u]
                    ns = jnp.where(rmask, zero, s * cB[u] + c_ * sB[u])
                    nc = jnp.where(rmask, one, c_ * cB[u] - s * sB[u])
                    plsc.addupdate(ebuf.at[e, t, pl.ds(kb + u * _L, _L)], ns)
                    plsc.addupdate(
                        ebuf.at[e, t, pl.ds(_H + kb + u * _L, _L)], nc)
                    out.append((ns, nc))
                return tuple(v[0] for v in out) + tuple(v[1] for v in out)

            fin = pl.loop(0, _CH, init_carry=init)(tbody)
            for u in range(4):
                st[pl.ds(kb + u * _L, _L)] = fin[u]
                st[pl.ds(_H + kb + u * _L, _L)] = fin[4 + u]

    @pl.loop(0, _NR)
    def _(r):
        for e in range(_NSLOT):
            c = r * _NSLOT + e
            drain_gather(e)
            chunk_add(e, c)

            pltpu.async_copy(ebuf.at[e], out_hbm.at[pl.ds(base + c * _CH, _CH)],
                             ssems[e])

            cf = c + _LOOKAHEAD
            ef = (e + _LOOKAHEAD) % _NSLOT

            @pl.when(jnp.logical_and(cf >= _NSLOT, cf < _NCHUNK))
            def _():
                drain_store(ef)

            @pl.when(cf < _NCHUNK)
            def _():
                fire_gather(cf, ef)

    for e in range(_NSLOT):
        drain_store(e)


def kernel(fv, rs, emb):
    ptab_np = np.concatenate(
        [_POS[1:2]] + [_POS[2 ** k:2 ** k + 1] for k in range(_NBIT)])
    ptab = jnp.asarray(ptab_np)
    rs_b = jnp.broadcast_to(rs[:, None], (_BATCH + 1, _L))
    mesh = plsc.VectorSubcoreMesh(
        core_axis_name="c", subcore_axis_name="s",
        num_cores=_NC, num_subcores=_NS,
    )
    k = pl.kernel(
        _body,
        out_type=jax.ShapeDtypeStruct((_TOTAL, _D), jnp.float32),
        mesh=mesh,
        scratch_types=[
            pltpu.VMEM((_TPW,), jnp.int32),
            pltpu.VMEM((_BATCH + 1, _L), jnp.int32),
            pltpu.VMEM((_TPW, _L), jnp.int32),
            pltpu.VMEM((1 + _NBIT, _D), jnp.float32),
            pltpu.VMEM((_D,), jnp.float32),
            pltpu.VMEM((_NSLOT, _CH, _D), jnp.float32),
        ] + [pltpu.SemaphoreType.DMA] * (2 * _NSLOT),
    )
    return k(fv, rs_b, emb, ptab)
